# Initial kernel scaffold; baseline (speedup 1.0000x reference)
import jax
import jax.numpy as jnp
from jax.experimental import pallas as pl


def kernel(atom_fea, edge_fea, edge_idx, W_e1, b_e1, W_a1, b_a1, W_e2, b_e2, W_a2, b_a2, W_e3, b_e3):
    raise NotImplementedError("write your pallas kernel here")


# speedup vs baseline: 4.2223x; 4.2223x over previous
import functools

import jax
import jax.numpy as jnp
from jax import lax
from jax.experimental import pallas as pl
from jax.experimental.pallas import tpu as pltpu
from jax.experimental.pallas import tpu_sc as plsc

N = 10000
E = 320000
D = 128
EPS = 1e-5

NC, NS = 2, 16
NW = NC * NS
EPW = E // NW
CH = 80
NCH = EPW // CH
N_PAD = 10240
RPT = N_PAD // NS
RB = 128

RT = 1000
NB = E // RT

@functools.cache
def _mesh():
    return plsc.VectorSubcoreMesh(
        core_axis_name="c", subcore_axis_name="s",
        num_cores=NC, num_subcores=NS)

f32 = jnp.float32


def _wid():
    return lax.axis_index("c") * NS + lax.axis_index("s")


def _sc_counts_body(dst_hbm, src_hbm, out_hbm, idxv, cntv):
    w = _wid()
    ones = jnp.ones((16,), f32)

    def do(idx_hbm, slot):
        pltpu.sync_copy(idx_hbm.at[w], idxv)

        def zero(i, c):
            cntv[pl.ds(i * 16, 16)] = jnp.zeros((16,), f32)
            return c
        lax.fori_loop(0, N // 16, zero, 0)

        def scat(r, c):
            for j in range(CH // 16):
                iv = idxv[r, pl.ds(j * 16, 16)]
                plsc.addupdate_scatter(cntv, [iv], ones)
            return c
        lax.fori_loop(0, NCH, scat, 0)
        pltpu.sync_copy(cntv, out_hbm.at[slot, w, 0])

    do(dst_hbm, 0)
    do(src_hbm, 1)


def _sc_counts(dst_c, src_c):
    return pl.kernel(
        _sc_counts_body,
        out_type=jax.ShapeDtypeStruct((2, NW, 1, N), f32),
        mesh=_mesh(),
        compiler_params=pltpu.CompilerParams(needs_layout_passes=False),
        scratch_types=[
            pltpu.VMEM((NCH, CH), jnp.int32),
            pltpu.VMEM((N,), f32),
        ],
    )(dst_c, src_c)


def _sc_gather_body(dsti, srci, pd_hbm, ps_hbm, g_hbm, idxd, idxs, bufd, bufs,
                    semd, sems):
    w = _wid()
    pltpu.sync_copy(dsti.at[w], idxd)
    pltpu.sync_copy(srci.at[w], idxs)

    def step(c, carry):
        cd = pltpu.async_copy(pd_hbm.at[idxd.at[c]], bufd, semd)
        cs = pltpu.async_copy(ps_hbm.at[idxs.at[c]], bufs, sems)
        cd.wait()
        cs.wait()

        def add_r(r, cc):
            for j in range(8):
                plsc.addupdate(bufd.at[r, pl.ds(j * 16, 16)],
                               bufs[r, pl.ds(j * 16, 16)])
            return cc
        lax.fori_loop(0, CH, add_r, 0)
        pltpu.sync_copy(bufd, g_hbm.at[pl.ds(w * EPW + c * CH, CH)])
        return carry
    lax.fori_loop(0, NCH, step, 0)


def _sc_gather(dst_c, src_c, pd, ps):
    return pl.kernel(
        _sc_gather_body,
        out_type=jax.ShapeDtypeStruct((E, D), f32),
        mesh=_mesh(),
        compiler_params=pltpu.CompilerParams(needs_layout_passes=False),
        scratch_types=[
            pltpu.VMEM((NCH, CH), jnp.int32),
            pltpu.VMEM((NCH, CH), jnp.int32),
            pltpu.VMEM((CH, D), f32),
            pltpu.VMEM((CH, D), f32),
            pltpu.SemaphoreType.DMA,
            pltpu.SemaphoreType.DMA,
        ],
    )(dst_c, src_c, pd, ps)


def _sc_scatter_body(z_hbm, dsti, out_hbm, idxd, zb, bounce, acc):
    cid = lax.axis_index("c")
    sid = lax.axis_index("s")
    w = cid * NS + sid
    pltpu.sync_copy(dsti.at[w], idxd)

    def zrow(r, c):
        for j in range(8):
            bounce[r, pl.ds(j * 16, 16)] = jnp.zeros((16,), f32)
        return c
    lax.fori_loop(0, RB, zrow, 0)
    for k in range(RPT // RB):
        pltpu.sync_copy(bounce, acc.at[pl.ds(sid * RPT + k * RB, RB)])
    plsc.subcore_barrier()

    def step(c, carry):
        pltpu.sync_copy(z_hbm.at[pl.ds(w * EPW + c * CH, CH)], zb)
        pltpu.sync_copy(zb, acc.at[idxd.at[c]], add=True)
        return carry
    lax.fori_loop(0, NCH, step, 0)
    plsc.subcore_barrier()

    for k in range(RPT // RB):
        pltpu.sync_copy(acc.at[pl.ds(sid * RPT + k * RB, RB)], bounce)
        pltpu.sync_copy(bounce, out_hbm.at[cid, pl.ds(sid * RPT + k * RB, RB)])


def _sc_scatter(z, dst_c):
    return pl.kernel(
        _sc_scatter_body,
        out_type=jax.ShapeDtypeStruct((NC, N_PAD, D), f32),
        mesh=_mesh(),
        compiler_params=pltpu.CompilerParams(needs_layout_passes=False),
        scratch_types=[
            pltpu.VMEM((NCH, CH), jnp.int32),
            pltpu.VMEM((CH, D), f32),
            pltpu.VMEM((RB, D), f32),
            pltpu.VMEM_SHARED((N_PAD, D), f32),
        ],
    )(z, dst_c)


def _softplus(t):
    return jnp.maximum(t, 0.0) + jnp.log1p(jnp.exp(-jnp.abs(t)))


def _tc_stats0_body(u_ref, out_ref):
    @pl.when(pl.program_id(0) == 0)
    def _():
        out_ref[...] = jnp.zeros_like(out_ref)
    u = u_ref[...]
    out_ref[0:1, :] += jnp.sum(u, 0, keepdims=True)
    out_ref[1:2, :] += jnp.sum(u * u, 0, keepdims=True)


def _tc_stats0(u):
    return pl.pallas_call(
        _tc_stats0_body,
        grid=(NB,),
        in_specs=[pl.BlockSpec((RT, D), lambda i: (i, 0))],
        out_specs=pl.BlockSpec((8, D), lambda i: (0, 0)),
        out_shape=jax.ShapeDtypeStruct((8, D), f32),
    )(u)


def _tc_a_body(u_ref, g_ref, wT_ref, bp_ref, out_ref):
    @pl.when(pl.program_id(0) == 0)
    def _():
        out_ref[...] = jnp.zeros_like(out_ref)
    y = (g_ref[...] + jnp.dot(u_ref[...], wT_ref[...],
                              preferred_element_type=f32) + bp_ref[...])
    out_ref[0:1, :] += jnp.sum(y, 0, keepdims=True)
    out_ref[1:2, :] += jnp.sum(y * y, 0, keepdims=True)


def _tc_a(u, g, wT, bp):
    return pl.pallas_call(
        _tc_a_body,
        grid=(NB,),
        in_specs=[
            pl.BlockSpec((RT, D), lambda i: (i, 0)),
            pl.BlockSpec((RT, D), lambda i: (i, 0)),
            pl.BlockSpec((D, D), lambda i: (0, 0)),
            pl.BlockSpec((1, D), lambda i: (0, 0)),
        ],
        out_specs=pl.BlockSpec((8, D), lambda i: (0, 0)),
        out_shape=jax.ShapeDtypeStruct((8, D), f32),
    )(u, g, wT, bp)


def _tc_b_body(u_ref, g_ref, wT_ref, bp_ref, cf_ref, z_ref, out_ref):
    @pl.when(pl.program_id(0) == 0)
    def _():
        out_ref[...] = jnp.zeros_like(out_ref)
    u = u_ref[...]
    y = (g_ref[...] + jnp.dot(u, wT_ref[...],
                              preferred_element_type=f32) + bp_ref[...])
    t = cf_ref[0:1, :] * y + cf_ref[1:2, :]
    z = cf_ref[2:3, :] * u + cf_ref[3:4, :] + _softplus(t)
    z_ref[...] = z
    out_ref[0:1, :] += jnp.sum(z, 0, keepdims=True)
    out_ref[1:2, :] += jnp.sum(z * z, 0, keepdims=True)


def _tc_b(u, g, wT, bp, cf):
    return pl.pallas_call(
        _tc_b_body,
        grid=(NB,),
        in_specs=[
            pl.BlockSpec((RT, D), lambda i: (i, 0)),
            pl.BlockSpec((RT, D), lambda i: (i, 0)),
            pl.BlockSpec((D, D), lambda i: (0, 0)),
            pl.BlockSpec((1, D), lambda i: (0, 0)),
            pl.BlockSpec((8, D), lambda i: (0, 0)),
        ],
        out_specs=[
            pl.BlockSpec((RT, D), lambda i: (i, 0)),
            pl.BlockSpec((8, D), lambda i: (0, 0)),
        ],
        out_shape=[
            jax.ShapeDtypeStruct((E, D), f32),
            jax.ShapeDtypeStruct((8, D), f32),
        ],
    )(u, g, wT, bp, cf)


def _proj(a, cd, cs, wdT, wsT):
    inv_e = 1.0 / E
    m_d = jnp.sum(cd * a, 0, keepdims=True) * inv_e
    q_d = jnp.sum(cd * a * a, 0, keepdims=True) * inv_e
    v_d = q_d - m_d * m_d
    a1d = lax.rsqrt(v_d + EPS)
    b1d = -m_d * a1d
    a2d = lax.rsqrt(v_d / (v_d + EPS) + EPS)
    Ad = a1d * a2d
    Bd = b1d * a2d
    m_s = jnp.sum(cs * a, 0, keepdims=True) * inv_e
    q_s = jnp.sum(cs * a * a, 0, keepdims=True) * inv_e
    v_s = q_s - m_s * m_s
    a1s = lax.rsqrt(v_s + EPS)
    b1s = -m_s * a1s
    a2s = lax.rsqrt(v_s / (v_s + EPS) + EPS)
    As = a1s * a2s
    Bs = b1s * a2s
    pd = jnp.dot(a * Ad, wdT, preferred_element_type=f32)
    ps = jnp.dot(a * As, wsT, preferred_element_type=f32)
    bfold = (jnp.dot(Bd, wdT, preferred_element_type=f32)
             + jnp.dot(Bs, wsT, preferred_element_type=f32))
    return pd, ps, bfold


def _tc_prep_body(a_ref, cd_ref, cs_ref, wdT_ref, wsT_ref,
                  pd_ref, ps_ref, aux_ref):
    pd, ps, bfold = _proj(a_ref[...], cd_ref[...], cs_ref[...],
                          wdT_ref[...], wsT_ref[...])
    pd_ref[...] = pd
    ps_ref[...] = ps
    aux_ref[...] = jnp.zeros_like(aux_ref)
    aux_ref[0:1, :] = bfold


def _tc_prep(a, cd, cs, wdT, wsT):
    return pl.pallas_call(
        _tc_prep_body,
        compiler_params=pltpu.CompilerParams(
            vmem_limit_bytes=100 * 1024 * 1024),
        out_shape=[
            jax.ShapeDtypeStruct((N, D), f32),
            jax.ShapeDtypeStruct((N, D), f32),
            jax.ShapeDtypeStruct((8, D), f32),
        ],
    )(a, cd, cs, wdT, wsT)


def _tc_atom_body(a_ref, S_ref, cd_ref, cf_ref, waT_ref, wpT_ref, ba_ref,
                  anew_ref):
    a = a_ref[...]
    cd = cd_ref[...]
    S = S_ref[...]
    au = cf_ref[0:1, :]
    cu = cf_ref[1:2, :]
    inv_n = 1.0 / N
    P = (au * S + cu * cd) / jnp.maximum(cd, 1.0)
    mP = jnp.sum(P, 0, keepdims=True) * inv_n
    vP = jnp.sum(P * P, 0, keepdims=True) * inv_n - mP * mP
    Pn = (P - mP) * lax.rsqrt(vP + EPS)
    Pn2 = Pn * lax.rsqrt(vP / (vP + EPS) + EPS)
    mA = jnp.sum(a, 0, keepdims=True) * inv_n
    vA = jnp.sum(a * a, 0, keepdims=True) * inv_n - mA * mA
    atn = (a - mA) * lax.rsqrt(vA + EPS)
    h = (jnp.dot(atn, waT_ref[...], preferred_element_type=f32)
         + jnp.dot(Pn2, wpT_ref[...], preferred_element_type=f32)
         + ba_ref[...])
    mh = jnp.sum(h, 0, keepdims=True) * inv_n
    vh = jnp.sum(h * h, 0, keepdims=True) * inv_n - mh * mh
    hn = (h - mh) * lax.rsqrt(vh + EPS)
    anu = a + _softplus(hn)
    ma = jnp.sum(anu, 0, keepdims=True) * inv_n
    va = jnp.sum(anu * anu, 0, keepdims=True) * inv_n - ma * ma
    anew_ref[...] = (anu - ma) * lax.rsqrt(va + EPS)


def _tc_atom(a, S, cd, cf, waT, wpT, ba):
    return pl.pallas_call(
        _tc_atom_body,
        out_shape=jax.ShapeDtypeStruct((N, D), f32),
        compiler_params=pltpu.CompilerParams(
            vmem_limit_bytes=100 * 1024 * 1024),
    )(a, S, cd, cf, waT, wpT, ba)


def _tc_final_body(z_ref, cf_ref, o_ref):
    o_ref[...] = cf_ref[0:1, :] * z_ref[...] + cf_ref[1:2, :]


def _tc_final(z, cf):
    return pl.pallas_call(
        _tc_final_body,
        grid=(NB,),
        in_specs=[
            pl.BlockSpec((RT, D), lambda i: (i, 0)),
            pl.BlockSpec((8, D), lambda i: (0, 0)),
        ],
        out_specs=pl.BlockSpec((RT, D), lambda i: (i, 0)),
        out_shape=jax.ShapeDtypeStruct((E, D), f32),
    )(z, cf)


def kernel(atom_fea, edge_fea, edge_idx, W_e1, b_e1, W_a1, b_a1,
           W_e2, b_e2, W_a2, b_a2, W_e3, b_e3):
    ei = edge_idx.astype(jnp.int32)
    dst = ei[:, 0]
    src = ei[:, 1]
    dst_c = dst.reshape(NW, NCH, CH)
    src_c = src.reshape(NW, NCH, CH)

    cnt_parts = _sc_counts(dst_c, src_c)
    cnt_d = cnt_parts[0].sum(axis=(0, 1))
    cnt_s = cnt_parts[1].sum(axis=(0, 1))
    cd_col = cnt_d[:, None]
    cs_col = cnt_s[:, None]

    s0 = _tc_stats0(edge_fea)
    mu = s0[0] / E
    vu = s0[1] / E - mu * mu

    u = edge_fea
    au = jnp.ones((D,), f32)
    cu = jnp.zeros((D,), f32)

    pd, ps, aux = _tc_prep(atom_fea, cd_col, cs_col,
                           W_e1[:, :D].T, W_e1[:, D:2 * D].T)
    bfold = aux[0]
    atom = atom_fea

    rounds = [(W_e1, b_e1, W_a1, b_a1, W_e2), (W_e2, b_e2, W_a2, b_a2, W_e3),
              (W_e3, b_e3, None, None, None)]
    for rnd, (W_e, b_e, W_a, b_a, W_e_next) in enumerate(rounds):
        W_u = W_e[:, 2 * D:]
        m_e = au * mu + cu
        v_e = au * au * vu
        ae2 = (v_e + EPS) ** -0.5
        A_u = au * ae2
        B_u = cu * ae2 - m_e * ae2
        WuT = (W_u * A_u[None, :]).T
        bp = (b_e + bfold + W_u @ B_u).reshape(1, D)

        g = _sc_gather(dst_c, src_c, pd, ps)
        sy = _tc_a(u, g, WuT, bp)
        my = sy[0] / E
        vy = sy[1] / E - my * my
        ay = (vy + EPS) ** -0.5
        by = -my * ay
        cf = (jnp.zeros((8, D), f32).at[0].set(ay).at[1].set(by)
              .at[2].set(au).at[3].set(cu))
        z, sz = _tc_b(u, g, WuT, bp, cf)
        mz = sz[0] / E
        vz = sz[1] / E - mz * mz
        au_n = (vz + EPS) ** -0.5
        cu_n = -mz * au_n

        if rnd == 2:
            cf2 = jnp.zeros((8, D), f32).at[0].set(au_n).at[1].set(cu_n)
            return _tc_final(z, cf2)

        Sp = _sc_scatter(z, dst_c)
        S = Sp[0, :N] + Sp[1, :N]
        cfa = jnp.zeros((8, D), f32).at[0].set(au_n).at[1].set(cu_n)
        atom = _tc_atom(atom, S, cd_col, cfa,
                        W_a[:, :D].T, W_a[:, D:].T, b_a.reshape(1, D))
        pd, ps, aux = _tc_prep(atom, cd_col, cs_col,
                               W_e_next[:, :D].T, W_e_next[:, D:2 * D].T)
        bfold = aux[0]
        u, au, cu, mu, vu = z, au_n, cu_n, mz, vz
